# Initial kernel scaffold; baseline (speedup 1.0000x reference)
import jax
import jax.numpy as jnp
from jax.experimental import pallas as pl


def kernel(u_feat, i_feat, uW0, ub0, uW1, ub1, uW2, ub2, uW3, ub3, iW0, ib0, iW1, ib1, iW2, ib2, iW3, ib3, Wr, Wgu, bgu, Wgi, bgi, Wu, bu, Wi, bi, Q, enc_edges, dec_edges):
    raise NotImplementedError("write your pallas kernel here")


# speedup vs baseline: 5.0824x; 5.0824x over previous
import functools

import jax
import jax.numpy as jnp
from jax import lax
from jax.experimental import pallas as pl
from jax.experimental.pallas import tpu as pltpu
from jax.experimental.pallas import tpu_sc as plsc

N_U = 10000
N_I = 10000
EMBED = 80
R = 5
MSG = EMBED // R
E_TOTAL = 640000
E_PER = E_TOTAL // R
P = 100000

NC = 2
NS = 16
EPT = E_PER // NS
CH = 80
NCH = EPT // CH
NODES_PT = 640
NODES_LAST = N_U - NODES_PT * (NS - 1)

BM = 1000
BMD = 2048


def _mlp_proj_body(x, w0, b0, w1, b1, w2, b2, w3, b3, wrf,
                   o0, o1, o2, o3, o4):
    h = jnp.maximum(jnp.dot(x[...], w0[...], preferred_element_type=jnp.float32) + b0[...], 0.0)
    h = jnp.maximum(jnp.dot(h, w1[...], preferred_element_type=jnp.float32) + b1[...], 0.0)
    h = jnp.maximum(jnp.dot(h, w2[...], preferred_element_type=jnp.float32) + b2[...], 0.0)
    h = jnp.maximum(jnp.dot(h, w3[...], preferred_element_type=jnp.float32) + b3[...], 0.0)
    p = jnp.dot(h, wrf[...], preferred_element_type=jnp.float32)
    for r, o in enumerate((o0, o1, o2, o3, o4)):
        o[...] = p[:, r * MSG:(r + 1) * MSG]


def _mlp_proj(x, ws, bs, wrf):
    n = x.shape[0]
    full = lambda a: pl.BlockSpec(a.shape, lambda i: (0,) * a.ndim)
    in_specs = [pl.BlockSpec((BM, x.shape[1]), lambda i: (i, 0))]
    args = [x]
    for w, b in zip(ws, bs):
        args += [w, b.reshape(1, -1)]
    args.append(wrf)
    in_specs += [full(a) for a in args[1:]]
    return pl.pallas_call(
        _mlp_proj_body,
        grid=(n // BM,),
        in_specs=in_specs,
        out_specs=[pl.BlockSpec((BM, MSG), lambda i: (i, 0))] * R,
        out_shape=[jax.ShapeDtypeStruct((n, MSG), jnp.float32)] * R,
    )(*args)


def _post_body(a0, a1, a2, a3, a4, deg, wg, bg, wl, bl, out):
    parts = []
    for r, a in enumerate((a0, a1, a2, a3, a4)):
        d = jnp.maximum(deg[...][:, r:r + 1], 1.0)
        parts.append(a[...] / d)
    e = jnp.maximum(jnp.concatenate(parts, axis=1), 0.0)
    e = jnp.maximum(jnp.dot(e, wg[...], preferred_element_type=jnp.float32) + bg[...], 0.0)
    out[...] = jnp.dot(e, wl[...], preferred_element_type=jnp.float32) + bl[...]


def _post(aggs, deg, wg, bg, wl, bl):
    n = deg.shape[0]
    full = lambda a: pl.BlockSpec(a.shape, lambda i: (0,) * a.ndim)
    args = list(aggs) + [deg, wg, bg.reshape(1, -1), wl, bl.reshape(1, -1)]
    in_specs = [pl.BlockSpec((BM, MSG), lambda i: (i, 0))] * (R + 1)
    in_specs += [full(a) for a in args[R + 1:]]
    return pl.pallas_call(
        _post_body,
        grid=(n // BM,),
        in_specs=in_specs,
        out_specs=pl.BlockSpec((BM, EMBED), lambda i: (i, 0)),
        out_shape=jax.ShapeDtypeStruct((n, EMBED), jnp.float32),
    )(*args)


def _decode_body(uu, ii, qf, out):
    t = jnp.dot(uu[...], qf[...], preferred_element_type=jnp.float32)
    iiv = ii[...]
    cols = []
    for r in range(R):
        cols.append(jnp.sum(t[:, r * EMBED:(r + 1) * EMBED] * iiv, axis=1, keepdims=True))
    out[...] = jnp.concatenate(cols, axis=1)


def _decode(uu, ii, qf):
    full = lambda a: pl.BlockSpec(a.shape, lambda i: (0,) * a.ndim)
    n = uu.shape[0]
    return pl.pallas_call(
        _decode_body,
        grid=(n // BMD,),
        in_specs=[pl.BlockSpec((BMD, EMBED), lambda i: (i, 0)),
                  pl.BlockSpec((BMD, EMBED), lambda i: (i, 0)),
                  full(qf)],
        out_specs=pl.BlockSpec((BMD, R), lambda i: (i, 0)),
        out_shape=jax.ShapeDtypeStruct((n, R), jnp.float32),
    )(uu, ii, qf)


def _enc_sc_body(up0, up1, up2, up3, up4, ip0, ip1, ip2, ip3, ip4, enc5,
                 au0, au1, au2, au3, au4, du,
                 ai0, ai1, ai2, ai3, ai4, di,
                 asp0, asp1, asp2, asp3, asp4, dsp,
                 zbuf, idxd, idxs, rows0, rows1, ones, sem0, sem1):
    c = lax.axis_index("c")
    s = lax.axis_index("s")
    asps = (asp0, asp1, asp2, asp3, asp4)

    def zrow(i, _):
        zbuf[i, :] = jnp.zeros((MSG,), jnp.float32)
        return 0

    lax.fori_loop(0, NODES_PT, zrow, 0)
    for sp in asps + (dsp,):
        @pl.when(s < NS - 1)
        def _():
            pltpu.sync_copy(zbuf, sp.at[pl.ds(s * NODES_PT, NODES_PT)])

        @pl.when(s == NS - 1)
        def _():
            pltpu.sync_copy(zbuf.at[pl.ds(0, NODES_LAST)],
                            sp.at[pl.ds((NS - 1) * NODES_PT, NODES_LAST)])
    plsc.subcore_barrier()

    def accumulate(dst_row, src_row, tabs):
        for r in range(R):
            onehot = jnp.where(lax.iota(jnp.int32, MSG) == r, 1.0, 0.0).astype(jnp.float32)

            def orow(i, _):
                ones[i, :] = onehot
                return 0

            lax.fori_loop(0, CH, orow, 0)
            pltpu.sync_copy(enc5.at[dst_row, r, s], idxd)
            pltpu.sync_copy(enc5.at[src_row, r, s], idxs)
            tab = tabs[r]
            asp = asps[r]
            pltpu.async_copy(tab.at[idxs.at[0]], rows0, sem0)

            def pair(k, _):
                j0 = 2 * k
                pltpu.make_async_copy(tab.at[idxs.at[j0]], rows0, sem0).wait()
                pltpu.async_copy(tab.at[idxs.at[j0 + 1]], rows1, sem1)
                pltpu.sync_copy(rows0, asp.at[idxd.at[j0]], add=True)
                pltpu.sync_copy(ones, dsp.at[idxd.at[j0]], add=True)
                pltpu.make_async_copy(tab.at[idxs.at[j0 + 1]], rows1, sem1).wait()

                @pl.when(k + 1 < NCH // 2)
                def _():
                    pltpu.async_copy(tab.at[idxs.at[j0 + 2]], rows0, sem0)

                pltpu.sync_copy(rows1, asp.at[idxd.at[j0 + 1]], add=True)
                pltpu.sync_copy(ones, dsp.at[idxd.at[j0 + 1]], add=True)
                return 0

            lax.fori_loop(0, NCH // 2, pair, 0)

    @pl.when(c == 0)
    def _():
        accumulate(0, 1, (ip0, ip1, ip2, ip3, ip4))

    @pl.when(c == 1)
    def _():
        accumulate(1, 0, (up0, up1, up2, up3, up4))

    plsc.subcore_barrier()

    for cc, outs in ((0, (au0, au1, au2, au3, au4, du)),
                     (1, (ai0, ai1, ai2, ai3, ai4, di))):
        @pl.when((c == cc) & (s < NS - 1))
        def _():
            sl = pl.ds(s * NODES_PT, NODES_PT)
            for sp, o in zip(asps + (dsp,), outs):
                pltpu.sync_copy(sp.at[sl], o.at[sl])

        @pl.when((c == cc) & (s == NS - 1))
        def _():
            sl = pl.ds((NS - 1) * NODES_PT, NODES_LAST)
            for sp, o in zip(asps + (dsp,), outs):
                pltpu.sync_copy(sp.at[sl], o.at[sl])


def _enc_sc(ups, ips, enc5):
    mesh = plsc.VectorSubcoreMesh(core_axis_name="c", subcore_axis_name="s",
                                  num_cores=NC, num_subcores=NS)
    out_type = [jax.ShapeDtypeStruct((N_U, MSG), jnp.float32)] * (R + 1) * 2
    scratch = ([pltpu.VMEM_SHARED((N_U, MSG), jnp.float32)] * R
               + [pltpu.VMEM_SHARED((N_U, MSG), jnp.float32)]
               + [pltpu.VMEM((NODES_PT, MSG), jnp.float32),
                  pltpu.VMEM((NCH, CH), jnp.int32),
                  pltpu.VMEM((NCH, CH), jnp.int32),
                  pltpu.VMEM((CH, MSG), jnp.float32),
                  pltpu.VMEM((CH, MSG), jnp.float32),
                  pltpu.VMEM((CH, MSG), jnp.float32),
                  pltpu.SemaphoreType.DMA,
                  pltpu.SemaphoreType.DMA])
    f = pl.kernel(_enc_sc_body, out_type=out_type, mesh=mesh,
                  scratch_types=scratch,
                  compiler_params=pltpu.CompilerParams(use_tc_tiling_on_sc=False))
    return f(*ups, *ips, enc5)


DEC_ROWS = 1280
P_PAD = DEC_ROWS * CH
DEC_RPT = DEC_ROWS // (NC * NS)


def _dec_sc_body(u_emb, i_emb, dec3, uu, ii,
                 idxu, idxi, rowsu, rowsi, semu, semi):
    c = lax.axis_index("c")
    s = lax.axis_index("s")
    w = s * NC + c
    pltpu.sync_copy(dec3.at[0, pl.ds(w * DEC_RPT, DEC_RPT)], idxu)
    pltpu.sync_copy(dec3.at[1, pl.ds(w * DEC_RPT, DEC_RPT)], idxi)

    def body(j, _):
        pltpu.async_copy(u_emb.at[idxu.at[j]], rowsu, semu)
        pltpu.async_copy(i_emb.at[idxi.at[j]], rowsi, semi)
        jj = w * DEC_RPT + j
        pltpu.make_async_copy(u_emb.at[idxu.at[j]], rowsu, semu).wait()
        pltpu.sync_copy(rowsu, uu.at[pl.ds(jj * CH, CH)])
        pltpu.make_async_copy(i_emb.at[idxi.at[j]], rowsi, semi).wait()
        pltpu.sync_copy(rowsi, ii.at[pl.ds(jj * CH, CH)])
        return 0

    lax.fori_loop(0, DEC_RPT, body, 0)


def _dec_sc(u_emb, i_emb, dec3):
    mesh = plsc.VectorSubcoreMesh(core_axis_name="c", subcore_axis_name="s",
                                  num_cores=NC, num_subcores=NS)
    out_type = [jax.ShapeDtypeStruct((P_PAD, EMBED), jnp.float32)] * 2
    scratch = [pltpu.VMEM((DEC_RPT, CH), jnp.int32),
               pltpu.VMEM((DEC_RPT, CH), jnp.int32),
               pltpu.VMEM((CH, EMBED), jnp.float32),
               pltpu.VMEM((CH, EMBED), jnp.float32),
               pltpu.SemaphoreType.DMA,
               pltpu.SemaphoreType.DMA]
    f = pl.kernel(_dec_sc_body, out_type=out_type, mesh=mesh,
                  scratch_types=scratch,
                  compiler_params=pltpu.CompilerParams(use_tc_tiling_on_sc=False))
    return f(u_emb, i_emb, dec3)


def kernel(u_feat, i_feat, uW0, ub0, uW1, ub1, uW2, ub2, uW3, ub3,
           iW0, ib0, iW1, ib1, iW2, ib2, iW3, ib3,
           Wr, Wgu, bgu, Wgi, bgi, Wu, bu, Wi, bi, Q,
           enc_edges, dec_edges):
    enc5 = enc_edges.astype(jnp.int32).reshape(2, R, NS, NCH, CH)
    dec_pad = jnp.concatenate(
        [dec_edges.astype(jnp.int32),
         jnp.zeros((2, P_PAD - P), jnp.int32)], axis=1)
    dec3 = dec_pad.reshape(2, DEC_ROWS, CH)
    wrf = jnp.transpose(Wr, (1, 0, 2)).reshape(EMBED, R * MSG)
    qf = jnp.transpose(Q, (1, 0, 2)).reshape(EMBED, R * EMBED)

    ups = _mlp_proj(u_feat, (uW0, uW1, uW2, uW3), (ub0, ub1, ub2, ub3), wrf)
    ips = _mlp_proj(i_feat, (iW0, iW1, iW2, iW3), (ib0, ib1, ib2, ib3), wrf)

    outs = _enc_sc(ups, ips, enc5)
    aggu, degu = outs[:R], outs[R]
    aggi, degi = outs[R + 1:2 * R + 1], outs[2 * R + 1]

    u_emb = _post(aggu, degu, Wgu, bgu, Wu, bu)
    i_emb = _post(aggi, degi, Wgi, bgi, Wi, bi)

    uu, ii = _dec_sc(u_emb, i_emb, dec3)
    return _decode(uu, ii, qf)[:P]


# speedup vs baseline: 6.3473x; 1.2489x over previous
import functools

import jax
import jax.numpy as jnp
from jax import lax
from jax.experimental import pallas as pl
from jax.experimental.pallas import tpu as pltpu
from jax.experimental.pallas import tpu_sc as plsc

N_U = 10000
N_I = 10000
EMBED = 80
R = 5
MSG = EMBED // R
E_TOTAL = 640000
E_PER = E_TOTAL // R
P = 100000

NC = 2
NS = 16
EPT = E_PER // NS
CH = 100
NCH = EPT // CH
RING = 8
NODES_PT = 640
NODES_LAST = N_U - NODES_PT * (NS - 1)

BM = 1000
BMD = 2048


def _mlp_proj_body(x, w0, b0, w1, b1, w2, b2, w3, b3, wrf,
                   o0, o1, o2, o3, o4):
    h = jnp.maximum(jnp.dot(x[...], w0[...], preferred_element_type=jnp.float32) + b0[...], 0.0)
    h = jnp.maximum(jnp.dot(h, w1[...], preferred_element_type=jnp.float32) + b1[...], 0.0)
    h = jnp.maximum(jnp.dot(h, w2[...], preferred_element_type=jnp.float32) + b2[...], 0.0)
    h = jnp.maximum(jnp.dot(h, w3[...], preferred_element_type=jnp.float32) + b3[...], 0.0)
    p = jnp.dot(h, wrf[...], preferred_element_type=jnp.float32)
    for r, o in enumerate((o0, o1, o2, o3, o4)):
        o[...] = p[:, r * MSG:(r + 1) * MSG]


def _mlp_proj(x, ws, bs, wrf):
    n = x.shape[0]
    full = lambda a: pl.BlockSpec(a.shape, lambda i: (0,) * a.ndim)
    in_specs = [pl.BlockSpec((BM, x.shape[1]), lambda i: (i, 0))]
    args = [x]
    for w, b in zip(ws, bs):
        args += [w, b.reshape(1, -1)]
    args.append(wrf)
    in_specs += [full(a) for a in args[1:]]
    return pl.pallas_call(
        _mlp_proj_body,
        grid=(n // BM,),
        in_specs=in_specs,
        out_specs=[pl.BlockSpec((BM, MSG), lambda i: (i, 0))] * R,
        out_shape=[jax.ShapeDtypeStruct((n, MSG), jnp.float32)] * R,
    )(*args)


def _post_body(a0, a1, a2, a3, a4, deg, wg, bg, wl, bl, out):
    parts = []
    for r, a in enumerate((a0, a1, a2, a3, a4)):
        d = jnp.maximum(deg[...][:, r:r + 1], 1.0)
        parts.append(a[...] / d)
    e = jnp.maximum(jnp.concatenate(parts, axis=1), 0.0)
    e = jnp.maximum(jnp.dot(e, wg[...], preferred_element_type=jnp.float32) + bg[...], 0.0)
    out[...] = jnp.dot(e, wl[...], preferred_element_type=jnp.float32) + bl[...]


def _post(aggs, deg, wg, bg, wl, bl):
    n = deg.shape[0]
    full = lambda a: pl.BlockSpec(a.shape, lambda i: (0,) * a.ndim)
    args = list(aggs) + [deg, wg, bg.reshape(1, -1), wl, bl.reshape(1, -1)]
    in_specs = [pl.BlockSpec((BM, MSG), lambda i: (i, 0))] * (R + 1)
    in_specs += [full(a) for a in args[R + 1:]]
    return pl.pallas_call(
        _post_body,
        grid=(n // BM,),
        in_specs=in_specs,
        out_specs=pl.BlockSpec((BM, EMBED), lambda i: (i, 0)),
        out_shape=jax.ShapeDtypeStruct((n, EMBED), jnp.float32),
    )(*args)


def _decode_body(uu, ii, qf, out):
    t = jnp.dot(uu[...], qf[...], preferred_element_type=jnp.float32)
    iiv = ii[...]
    cols = []
    for r in range(R):
        cols.append(jnp.sum(t[:, r * EMBED:(r + 1) * EMBED] * iiv, axis=1, keepdims=True))
    out[...] = jnp.concatenate(cols, axis=1)


def _decode(uu, ii, qf):
    full = lambda a: pl.BlockSpec(a.shape, lambda i: (0,) * a.ndim)
    n = uu.shape[0]
    return pl.pallas_call(
        _decode_body,
        grid=(n // BMD,),
        in_specs=[pl.BlockSpec((BMD, EMBED), lambda i: (i, 0)),
                  pl.BlockSpec((BMD, EMBED), lambda i: (i, 0)),
                  full(qf)],
        out_specs=pl.BlockSpec((BMD, R), lambda i: (i, 0)),
        out_shape=jax.ShapeDtypeStruct((n, R), jnp.float32),
    )(uu, ii, qf)


def _enc_sc_body(up0, up1, up2, up3, up4, ip0, ip1, ip2, ip3, ip4, enc5,
                 au0, au1, au2, au3, au4, du,
                 ai0, ai1, ai2, ai3, ai4, di,
                 asp0, asp1, asp2, asp3, asp4, dsp,
                 zbuf, idxd, idxs, rows, ones, gsem, ssem):
    c = lax.axis_index("c")
    s = lax.axis_index("s")
    asps = (asp0, asp1, asp2, asp3, asp4)

    def zrow(i, _):
        zbuf[i, :] = jnp.zeros((MSG,), jnp.float32)
        return 0

    lax.fori_loop(0, NODES_PT, zrow, 0)
    for sp in asps + (dsp,):
        @pl.when(s < NS - 1)
        def _():
            pltpu.sync_copy(zbuf, sp.at[pl.ds(s * NODES_PT, NODES_PT)])

        @pl.when(s == NS - 1)
        def _():
            pltpu.sync_copy(zbuf.at[pl.ds(0, NODES_LAST)],
                            sp.at[pl.ds((NS - 1) * NODES_PT, NODES_LAST)])
    plsc.subcore_barrier()

    def accumulate(dst_row, src_row, tabs):
        for r in range(R):
            onehot = jnp.where(lax.iota(jnp.int32, MSG) == r, 1.0, 0.0).astype(jnp.float32)

            def orow(i, _):
                ones[i, :] = onehot
                return 0

            lax.fori_loop(0, CH, orow, 0)
            pltpu.sync_copy(enc5.at[dst_row, r, s], idxd)
            pltpu.sync_copy(enc5.at[src_row, r, s], idxs)
            tab = tabs[r]
            asp = asps[r]
            for b in range(RING // 2):
                pltpu.async_copy(tab.at[idxs.at[b]], rows.at[b], gsem.at[b])

            def group(kk, _):
                for b in range(RING):
                    j = kk * RING + b
                    bn = (b + RING // 2) % RING

                    @pl.when(j >= RING // 2)
                    def _():
                        pltpu.make_async_copy(
                            rows.at[bn], asp.at[idxd.at[j - RING // 2]],
                            ssem.at[bn]).wait()
                        pltpu.make_async_copy(
                            ones, dsp.at[idxd.at[j - RING // 2]],
                            ssem.at[bn]).wait()

                    @pl.when(j + RING // 2 < NCH)
                    def _():
                        pltpu.async_copy(tab.at[idxs.at[j + RING // 2]],
                                         rows.at[bn], gsem.at[bn])

                    pltpu.make_async_copy(tab.at[idxs.at[j]], rows.at[b],
                                          gsem.at[b]).wait()
                    pltpu.async_copy(rows.at[b], asp.at[idxd.at[j]],
                                     ssem.at[b], add=True)
                    pltpu.async_copy(ones, dsp.at[idxd.at[j]],
                                     ssem.at[b], add=True)
                return 0

            lax.fori_loop(0, NCH // RING, group, 0)
            for b in range(RING // 2, RING):
                j = NCH - RING + b
                pltpu.make_async_copy(rows.at[b], asp.at[idxd.at[j]],
                                      ssem.at[b]).wait()
                pltpu.make_async_copy(ones, dsp.at[idxd.at[j]],
                                      ssem.at[b]).wait()

    @pl.when(c == 0)
    def _():
        accumulate(0, 1, (ip0, ip1, ip2, ip3, ip4))

    @pl.when(c == 1)
    def _():
        accumulate(1, 0, (up0, up1, up2, up3, up4))

    plsc.subcore_barrier()

    for cc, outs in ((0, (au0, au1, au2, au3, au4, du)),
                     (1, (ai0, ai1, ai2, ai3, ai4, di))):
        @pl.when((c == cc) & (s < NS - 1))
        def _():
            sl = pl.ds(s * NODES_PT, NODES_PT)
            for sp, o in zip(asps + (dsp,), outs):
                pltpu.sync_copy(sp.at[sl], o.at[sl])

        @pl.when((c == cc) & (s == NS - 1))
        def _():
            sl = pl.ds((NS - 1) * NODES_PT, NODES_LAST)
            for sp, o in zip(asps + (dsp,), outs):
                pltpu.sync_copy(sp.at[sl], o.at[sl])


def _enc_sc(ups, ips, enc5):
    mesh = plsc.VectorSubcoreMesh(core_axis_name="c", subcore_axis_name="s",
                                  num_cores=NC, num_subcores=NS)
    out_type = [jax.ShapeDtypeStruct((N_U, MSG), jnp.float32)] * (R + 1) * 2
    scratch = ([pltpu.VMEM_SHARED((N_U, MSG), jnp.float32)] * R
               + [pltpu.VMEM_SHARED((N_U, MSG), jnp.float32)]
               + [pltpu.VMEM((NODES_PT, MSG), jnp.float32),
                  pltpu.VMEM((NCH, CH), jnp.int32),
                  pltpu.VMEM((NCH, CH), jnp.int32),
                  pltpu.VMEM((RING, CH, MSG), jnp.float32),
                  pltpu.VMEM((CH, MSG), jnp.float32),
                  pltpu.SemaphoreType.DMA((RING,)),
                  pltpu.SemaphoreType.DMA((RING,))])
    f = pl.kernel(_enc_sc_body, out_type=out_type, mesh=mesh,
                  scratch_types=scratch,
                  compiler_params=pltpu.CompilerParams(use_tc_tiling_on_sc=False))
    return f(*ups, *ips, enc5)


DCH = 80
DEC_ROWS = 1280
P_PAD = DEC_ROWS * DCH
DEC_RPT = DEC_ROWS // (NC * NS)
DRING = 4


def _dec_sc_body(u_emb, i_emb, dec3, uu, ii,
                 idxu, idxi, rub, rib, gsu, gsi, wsu, wsi):
    c = lax.axis_index("c")
    s = lax.axis_index("s")
    w = s * NC + c
    pltpu.sync_copy(dec3.at[0, pl.ds(w * DEC_RPT, DEC_RPT)], idxu)
    pltpu.sync_copy(dec3.at[1, pl.ds(w * DEC_RPT, DEC_RPT)], idxi)
    for b in range(DRING // 2):
        pltpu.async_copy(u_emb.at[idxu.at[b]], rub.at[b], gsu.at[b])
        pltpu.async_copy(i_emb.at[idxi.at[b]], rib.at[b], gsi.at[b])

    def group(kk, _):
        for b in range(DRING):
            j = kk * DRING + b
            bn = (b + DRING // 2) % DRING
            jj = w * DEC_RPT + j

            @pl.when(j >= DRING // 2)
            def _():
                jo = (w * DEC_RPT + j - DRING // 2) * DCH
                pltpu.make_async_copy(rub.at[bn], uu.at[pl.ds(jo, DCH)],
                                      wsu.at[bn]).wait()
                pltpu.make_async_copy(rib.at[bn], ii.at[pl.ds(jo, DCH)],
                                      wsi.at[bn]).wait()

            @pl.when(j + DRING // 2 < DEC_RPT)
            def _():
                pltpu.async_copy(u_emb.at[idxu.at[j + DRING // 2]],
                                 rub.at[bn], gsu.at[bn])
                pltpu.async_copy(i_emb.at[idxi.at[j + DRING // 2]],
                                 rib.at[bn], gsi.at[bn])

            pltpu.make_async_copy(u_emb.at[idxu.at[j]], rub.at[b],
                                  gsu.at[b]).wait()
            pltpu.make_async_copy(i_emb.at[idxi.at[j]], rib.at[b],
                                  gsi.at[b]).wait()
            pltpu.async_copy(rub.at[b], uu.at[pl.ds(jj * DCH, DCH)],
                             wsu.at[b])
            pltpu.async_copy(rib.at[b], ii.at[pl.ds(jj * DCH, DCH)],
                             wsi.at[b])
        return 0

    lax.fori_loop(0, DEC_RPT // DRING, group, 0)
    for b in range(DRING // 2, DRING):
        jo = (w * DEC_RPT + DEC_RPT - DRING + b) * DCH
        pltpu.make_async_copy(rub.at[b], uu.at[pl.ds(jo, DCH)],
                              wsu.at[b]).wait()
        pltpu.make_async_copy(rib.at[b], ii.at[pl.ds(jo, DCH)],
                              wsi.at[b]).wait()


def _dec_sc(u_emb, i_emb, dec3):
    mesh = plsc.VectorSubcoreMesh(core_axis_name="c", subcore_axis_name="s",
                                  num_cores=NC, num_subcores=NS)
    out_type = [jax.ShapeDtypeStruct((P_PAD, EMBED), jnp.float32)] * 2
    scratch = [pltpu.VMEM((DEC_RPT, DCH), jnp.int32),
               pltpu.VMEM((DEC_RPT, DCH), jnp.int32),
               pltpu.VMEM((DRING, DCH, EMBED), jnp.float32),
               pltpu.VMEM((DRING, DCH, EMBED), jnp.float32),
               pltpu.SemaphoreType.DMA((DRING,)),
               pltpu.SemaphoreType.DMA((DRING,)),
               pltpu.SemaphoreType.DMA((DRING,)),
               pltpu.SemaphoreType.DMA((DRING,))]
    f = pl.kernel(_dec_sc_body, out_type=out_type, mesh=mesh,
                  scratch_types=scratch,
                  compiler_params=pltpu.CompilerParams(use_tc_tiling_on_sc=False))
    return f(u_emb, i_emb, dec3)


def kernel(u_feat, i_feat, uW0, ub0, uW1, ub1, uW2, ub2, uW3, ub3,
           iW0, ib0, iW1, ib1, iW2, ib2, iW3, ib3,
           Wr, Wgu, bgu, Wgi, bgi, Wu, bu, Wi, bi, Q,
           enc_edges, dec_edges):
    enc5 = enc_edges.astype(jnp.int32).reshape(2, R, NS, NCH, CH)
    dec_pad = jnp.concatenate(
        [dec_edges.astype(jnp.int32),
         jnp.zeros((2, P_PAD - P), jnp.int32)], axis=1)
    dec3 = dec_pad.reshape(2, DEC_ROWS, DCH)
    wrf = jnp.transpose(Wr, (1, 0, 2)).reshape(EMBED, R * MSG)
    qf = jnp.transpose(Q, (1, 0, 2)).reshape(EMBED, R * EMBED)

    ups = _mlp_proj(u_feat, (uW0, uW1, uW2, uW3), (ub0, ub1, ub2, ub3), wrf)
    ips = _mlp_proj(i_feat, (iW0, iW1, iW2, iW3), (ib0, ib1, ib2, ib3), wrf)

    outs = _enc_sc(ups, ips, enc5)
    aggu, degu = outs[:R], outs[R]
    aggi, degi = outs[R + 1:2 * R + 1], outs[2 * R + 1]

    u_emb = _post(aggu, degu, Wgu, bgu, Wu, bu)
    i_emb = _post(aggi, degi, Wgi, bgi, Wi, bi)

    uu, ii = _dec_sc(u_emb, i_emb, dec3)
    return _decode(uu, ii, qf)[:P]


# speedup vs baseline: 8.5699x; 1.3502x over previous
import functools

import jax
import jax.numpy as jnp
from jax import lax
from jax.experimental import pallas as pl
from jax.experimental.pallas import tpu as pltpu
from jax.experimental.pallas import tpu_sc as plsc

N_U = 10000
N_I = 10000
EMBED = 80
R = 5
MSG = EMBED // R
E_TOTAL = 640000
E_PER = E_TOTAL // R
P = 100000

NC = 2
NS = 16
EPT = E_PER // NS
CH = 100
NCH = EPT // CH
RING = 8
NODES_PT = 640
NODES_LAST = N_U - NODES_PT * (NS - 1)

BM = 1000
BMD = 2048


def _mlp_proj_body(x, w0, b0, w1, b1, w2, b2, w3, b3, wrf,
                   o0, o1, o2, o3, o4):
    h = jnp.maximum(jnp.dot(x[...], w0[...], preferred_element_type=jnp.float32) + b0[...], 0.0)
    h = jnp.maximum(jnp.dot(h, w1[...], preferred_element_type=jnp.float32) + b1[...], 0.0)
    h = jnp.maximum(jnp.dot(h, w2[...], preferred_element_type=jnp.float32) + b2[...], 0.0)
    h = jnp.maximum(jnp.dot(h, w3[...], preferred_element_type=jnp.float32) + b3[...], 0.0)
    p = jnp.dot(h, wrf[...], preferred_element_type=jnp.float32)
    for r, o in enumerate((o0, o1, o2, o3, o4)):
        o[...] = p[:, r * MSG:(r + 1) * MSG]


def _mlp_proj(x, ws, bs, wrf):
    n = x.shape[0]
    full = lambda a: pl.BlockSpec(a.shape, lambda i: (0,) * a.ndim)
    in_specs = [pl.BlockSpec((BM, x.shape[1]), lambda i: (i, 0))]
    args = [x]
    for w, b in zip(ws, bs):
        args += [w, b.reshape(1, -1)]
    args.append(wrf)
    in_specs += [full(a) for a in args[1:]]
    return pl.pallas_call(
        _mlp_proj_body,
        grid=(n // BM,),
        in_specs=in_specs,
        out_specs=[pl.BlockSpec((BM, MSG), lambda i: (i, 0))] * R,
        out_shape=[jax.ShapeDtypeStruct((n, MSG), jnp.float32)] * R,
    )(*args)


def _post_body(a0, a1, a2, a3, a4, deg, wg, bg, wl, bl, out):
    parts = []
    for r, a in enumerate((a0, a1, a2, a3, a4)):
        d = jnp.maximum(deg[...][:, r:r + 1], 1.0)
        parts.append(a[...] / d)
    e = jnp.maximum(jnp.concatenate(parts, axis=1), 0.0)
    e = jnp.maximum(jnp.dot(e, wg[...], preferred_element_type=jnp.float32) + bg[...], 0.0)
    out[...] = jnp.dot(e, wl[...], preferred_element_type=jnp.float32) + bl[...]


def _post(aggs, deg, wg, bg, wl, bl):
    n = deg.shape[0]
    full = lambda a: pl.BlockSpec(a.shape, lambda i: (0,) * a.ndim)
    args = list(aggs) + [deg, wg, bg.reshape(1, -1), wl, bl.reshape(1, -1)]
    in_specs = [pl.BlockSpec((BM, MSG), lambda i: (i, 0))] * (R + 1)
    in_specs += [full(a) for a in args[R + 1:]]
    return pl.pallas_call(
        _post_body,
        grid=(n // BM,),
        in_specs=in_specs,
        out_specs=pl.BlockSpec((BM, EMBED), lambda i: (i, 0)),
        out_shape=jax.ShapeDtypeStruct((n, EMBED), jnp.float32),
    )(*args)


def _decode_body(uu, ii, qf, em, out):
    t = jnp.dot(uu[...], qf[...], preferred_element_type=jnp.float32)
    iiv = ii[...]
    m = t * jnp.concatenate([iiv] * R, axis=1)
    out[...] = jnp.dot(m, em[...], preferred_element_type=jnp.float32)


def _decode(uu, ii, qf, em):
    full = lambda a: pl.BlockSpec(a.shape, lambda i: (0,) * a.ndim)
    n = uu.shape[0]
    return pl.pallas_call(
        _decode_body,
        grid=(n // BMD,),
        in_specs=[pl.BlockSpec((BMD, EMBED), lambda i: (i, 0)),
                  pl.BlockSpec((BMD, EMBED), lambda i: (i, 0)),
                  full(qf), full(em)],
        out_specs=pl.BlockSpec((BMD, R), lambda i: (i, 0)),
        out_shape=jax.ShapeDtypeStruct((n, R), jnp.float32),
    )(uu, ii, qf, em)


def _enc_sc_body(up0, up1, up2, up3, up4, ip0, ip1, ip2, ip3, ip4, enc5,
                 au0, au1, au2, au3, au4, du,
                 ai0, ai1, ai2, ai3, ai4, di,
                 asp0, asp1, asp2, asp3, asp4, dsp,
                 zbuf, idxd, idxs, rows, ones, gsem, ssem):
    c = lax.axis_index("c")
    s = lax.axis_index("s")
    asps = (asp0, asp1, asp2, asp3, asp4)

    def zrow(i, _):
        zbuf[i, :] = jnp.zeros((MSG,), jnp.float32)
        return 0

    lax.fori_loop(0, NODES_PT, zrow, 0)
    for sp in asps + (dsp,):
        @pl.when(s < NS - 1)
        def _():
            pltpu.sync_copy(zbuf, sp.at[pl.ds(s * NODES_PT, NODES_PT)])

        @pl.when(s == NS - 1)
        def _():
            pltpu.sync_copy(zbuf.at[pl.ds(0, NODES_LAST)],
                            sp.at[pl.ds((NS - 1) * NODES_PT, NODES_LAST)])
    plsc.subcore_barrier()

    def accumulate(dst_row, src_row, tabs):
        for r in range(R):
            onehot = jnp.where(lax.iota(jnp.int32, MSG) == r, 1.0, 0.0).astype(jnp.float32)

            def orow(i, _):
                ones[i, :] = onehot
                return 0

            lax.fori_loop(0, CH, orow, 0)
            pltpu.sync_copy(enc5.at[dst_row, r, s], idxd)
            pltpu.sync_copy(enc5.at[src_row, r, s], idxs)
            tab = tabs[r]
            asp = asps[r]
            for b in range(RING // 2):
                pltpu.async_copy(tab.at[idxs.at[b]], rows.at[b], gsem.at[b])

            def group(kk, _):
                for b in range(RING):
                    j = kk * RING + b
                    bn = (b + RING // 2) % RING

                    @pl.when(j >= RING // 2)
                    def _():
                        pltpu.make_async_copy(
                            rows.at[bn], asp.at[idxd.at[j - RING // 2]],
                            ssem.at[bn]).wait()
                        pltpu.make_async_copy(
                            ones, dsp.at[idxd.at[j - RING // 2]],
                            ssem.at[bn]).wait()

                    @pl.when(j + RING // 2 < NCH)
                    def _():
                        pltpu.async_copy(tab.at[idxs.at[j + RING // 2]],
                                         rows.at[bn], gsem.at[bn])

                    pltpu.make_async_copy(tab.at[idxs.at[j]], rows.at[b],
                                          gsem.at[b]).wait()
                    pltpu.async_copy(rows.at[b], asp.at[idxd.at[j]],
                                     ssem.at[b], add=True)
                    pltpu.async_copy(ones, dsp.at[idxd.at[j]],
                                     ssem.at[b], add=True)
                return 0

            lax.fori_loop(0, NCH // RING, group, 0)
            for b in range(RING // 2, RING):
                j = NCH - RING + b
                pltpu.make_async_copy(rows.at[b], asp.at[idxd.at[j]],
                                      ssem.at[b]).wait()
                pltpu.make_async_copy(ones, dsp.at[idxd.at[j]],
                                      ssem.at[b]).wait()

    @pl.when(c == 0)
    def _():
        accumulate(0, 1, (ip0, ip1, ip2, ip3, ip4))

    @pl.when(c == 1)
    def _():
        accumulate(1, 0, (up0, up1, up2, up3, up4))

    plsc.subcore_barrier()

    for cc, outs in ((0, (au0, au1, au2, au3, au4, du)),
                     (1, (ai0, ai1, ai2, ai3, ai4, di))):
        @pl.when((c == cc) & (s < NS - 1))
        def _():
            sl = pl.ds(s * NODES_PT, NODES_PT)
            for sp, o in zip(asps + (dsp,), outs):
                pltpu.sync_copy(sp.at[sl], o.at[sl])

        @pl.when((c == cc) & (s == NS - 1))
        def _():
            sl = pl.ds((NS - 1) * NODES_PT, NODES_LAST)
            for sp, o in zip(asps + (dsp,), outs):
                pltpu.sync_copy(sp.at[sl], o.at[sl])


def _enc_sc(ups, ips, enc5):
    mesh = plsc.VectorSubcoreMesh(core_axis_name="c", subcore_axis_name="s",
                                  num_cores=NC, num_subcores=NS)
    out_type = [jax.ShapeDtypeStruct((N_U, MSG), jnp.float32)] * (R + 1) * 2
    scratch = ([pltpu.VMEM_SHARED((N_U, MSG), jnp.float32)] * R
               + [pltpu.VMEM_SHARED((N_U, MSG), jnp.float32)]
               + [pltpu.VMEM((NODES_PT, MSG), jnp.float32),
                  pltpu.VMEM((NCH, CH), jnp.int32),
                  pltpu.VMEM((NCH, CH), jnp.int32),
                  pltpu.VMEM((RING, CH, MSG), jnp.float32),
                  pltpu.VMEM((CH, MSG), jnp.float32),
                  pltpu.SemaphoreType.DMA((RING,)),
                  pltpu.SemaphoreType.DMA((RING,))])
    f = pl.kernel(_enc_sc_body, out_type=out_type, mesh=mesh,
                  scratch_types=scratch,
                  compiler_params=pltpu.CompilerParams(use_tc_tiling_on_sc=False))
    return f(*ups, *ips, enc5)


DCH = 80
DEC_ROWS = 1280
P_PAD = DEC_ROWS * DCH
DEC_RPT = DEC_ROWS // (NC * NS)
DRING = 4


def _dec_sc_body(u_emb, i_emb, dec3, uu, ii,
                 idxu, idxi, rub, rib, gsu, gsi, wsu, wsi):
    c = lax.axis_index("c")
    s = lax.axis_index("s")
    w = s * NC + c
    pltpu.sync_copy(dec3.at[0, pl.ds(w * DEC_RPT, DEC_RPT)], idxu)
    pltpu.sync_copy(dec3.at[1, pl.ds(w * DEC_RPT, DEC_RPT)], idxi)
    for b in range(DRING // 2):
        pltpu.async_copy(u_emb.at[idxu.at[b]], rub.at[b], gsu.at[b])
        pltpu.async_copy(i_emb.at[idxi.at[b]], rib.at[b], gsi.at[b])

    def group(kk, _):
        for b in range(DRING):
            j = kk * DRING + b
            bn = (b + DRING // 2) % DRING
            jj = w * DEC_RPT + j

            @pl.when(j >= DRING // 2)
            def _():
                jo = (w * DEC_RPT + j - DRING // 2) * DCH
                pltpu.make_async_copy(rub.at[bn], uu.at[pl.ds(jo, DCH)],
                                      wsu.at[bn]).wait()
                pltpu.make_async_copy(rib.at[bn], ii.at[pl.ds(jo, DCH)],
                                      wsi.at[bn]).wait()

            @pl.when(j + DRING // 2 < DEC_RPT)
            def _():
                pltpu.async_copy(u_emb.at[idxu.at[j + DRING // 2]],
                                 rub.at[bn], gsu.at[bn])
                pltpu.async_copy(i_emb.at[idxi.at[j + DRING // 2]],
                                 rib.at[bn], gsi.at[bn])

            pltpu.make_async_copy(u_emb.at[idxu.at[j]], rub.at[b],
                                  gsu.at[b]).wait()
            pltpu.make_async_copy(i_emb.at[idxi.at[j]], rib.at[b],
                                  gsi.at[b]).wait()
            pltpu.async_copy(rub.at[b], uu.at[pl.ds(jj * DCH, DCH)],
                             wsu.at[b])
            pltpu.async_copy(rib.at[b], ii.at[pl.ds(jj * DCH, DCH)],
                             wsi.at[b])
        return 0

    lax.fori_loop(0, DEC_RPT // DRING, group, 0)
    for b in range(DRING // 2, DRING):
        jo = (w * DEC_RPT + DEC_RPT - DRING + b) * DCH
        pltpu.make_async_copy(rub.at[b], uu.at[pl.ds(jo, DCH)],
                              wsu.at[b]).wait()
        pltpu.make_async_copy(rib.at[b], ii.at[pl.ds(jo, DCH)],
                              wsi.at[b]).wait()


def _dec_sc(u_emb, i_emb, dec3):
    mesh = plsc.VectorSubcoreMesh(core_axis_name="c", subcore_axis_name="s",
                                  num_cores=NC, num_subcores=NS)
    out_type = [jax.ShapeDtypeStruct((P_PAD, EMBED), jnp.float32)] * 2
    scratch = [pltpu.VMEM((DEC_RPT, DCH), jnp.int32),
               pltpu.VMEM((DEC_RPT, DCH), jnp.int32),
               pltpu.VMEM((DRING, DCH, EMBED), jnp.float32),
               pltpu.VMEM((DRING, DCH, EMBED), jnp.float32),
               pltpu.SemaphoreType.DMA((DRING,)),
               pltpu.SemaphoreType.DMA((DRING,)),
               pltpu.SemaphoreType.DMA((DRING,)),
               pltpu.SemaphoreType.DMA((DRING,))]
    f = pl.kernel(_dec_sc_body, out_type=out_type, mesh=mesh,
                  scratch_types=scratch,
                  compiler_params=pltpu.CompilerParams(use_tc_tiling_on_sc=False))
    return f(u_emb, i_emb, dec3)


def kernel(u_feat, i_feat, uW0, ub0, uW1, ub1, uW2, ub2, uW3, ub3,
           iW0, ib0, iW1, ib1, iW2, ib2, iW3, ib3,
           Wr, Wgu, bgu, Wgi, bgi, Wu, bu, Wi, bi, Q,
           enc_edges, dec_edges):
    enc5 = enc_edges.astype(jnp.int32).reshape(2, R, NS, NCH, CH)
    dec_pad = jnp.concatenate(
        [dec_edges.astype(jnp.int32),
         jnp.zeros((2, P_PAD - P), jnp.int32)], axis=1)
    dec3 = dec_pad.reshape(2, DEC_ROWS, DCH)
    wrf = jnp.transpose(Wr, (1, 0, 2)).reshape(EMBED, R * MSG)
    qf = jnp.transpose(Q, (1, 0, 2)).reshape(EMBED, R * EMBED)
    em = (jnp.arange(R * EMBED)[:, None] // EMBED
          == jnp.arange(R)[None, :]).astype(jnp.float32)

    ups = _mlp_proj(u_feat, (uW0, uW1, uW2, uW3), (ub0, ub1, ub2, ub3), wrf)
    ips = _mlp_proj(i_feat, (iW0, iW1, iW2, iW3), (ib0, ib1, ib2, ib3), wrf)

    outs = _enc_sc(ups, ips, enc5)
    aggu, degu = outs[:R], outs[R]
    aggi, degi = outs[R + 1:2 * R + 1], outs[2 * R + 1]

    u_emb = _post(aggu, degu, Wgu, bgu, Wu, bu)
    i_emb = _post(aggi, degi, Wgi, bgi, Wi, bi)

    uu, ii = _dec_sc(u_emb, i_emb, dec3)
    return _decode(uu, ii, qf, em)[:P]


# speedup vs baseline: 9.7181x; 1.1340x over previous
import functools

import jax
import jax.numpy as jnp
from jax import lax
from jax.experimental import pallas as pl
from jax.experimental.pallas import tpu as pltpu
from jax.experimental.pallas import tpu_sc as plsc

N_U = 10000
N_I = 10000
EMBED = 80
R = 5
MSG = EMBED // R
E_TOTAL = 640000
E_PER = E_TOTAL // R
P = 100000

NC = 2
NS = 16
EPT = E_PER // NS
CH = 100
NCH = EPT // CH
RING = 8
NODES_PT = 640
NODES_LAST = N_U - NODES_PT * (NS - 1)

BM = 1000
BMD = 2048
LANE = 128
DEG0 = EMBED


def _mlp_proj_body(x, w0, b0, w1, b1, w2, b2, w3, b3, wrf,
                   o0, o1, o2, o3, o4):
    h = jnp.maximum(jnp.dot(x[...], w0[...], preferred_element_type=jnp.float32) + b0[...], 0.0)
    h = jnp.maximum(jnp.dot(h, w1[...], preferred_element_type=jnp.float32) + b1[...], 0.0)
    h = jnp.maximum(jnp.dot(h, w2[...], preferred_element_type=jnp.float32) + b2[...], 0.0)
    h = jnp.maximum(jnp.dot(h, w3[...], preferred_element_type=jnp.float32) + b3[...], 0.0)
    p = jnp.dot(h, wrf[...], preferred_element_type=jnp.float32)
    for r, o in enumerate((o0, o1, o2, o3, o4)):
        o[...] = p[:, r * MSG:(r + 1) * MSG]


def _mlp_proj(x, ws, bs, wrf):
    n = x.shape[0]
    full = lambda a: pl.BlockSpec(a.shape, lambda i: (0,) * a.ndim)
    in_specs = [pl.BlockSpec((BM, x.shape[1]), lambda i: (i, 0))]
    args = [x]
    for w, b in zip(ws, bs):
        args += [w, b.reshape(1, -1)]
    args.append(wrf)
    in_specs += [full(a) for a in args[1:]]
    return pl.pallas_call(
        _mlp_proj_body,
        grid=(n // BM,),
        in_specs=in_specs,
        out_specs=[pl.BlockSpec((BM, MSG), lambda i: (i, 0))] * R,
        out_shape=[jax.ShapeDtypeStruct((n, MSG), jnp.float32)] * R,
    )(*args)


def _post_body(ab, wg, bg, wl, bl, out):
    abv = ab[...]
    parts = []
    for r in range(R):
        d = jnp.maximum(abv[:, DEG0 + r:DEG0 + r + 1], 1.0)
        parts.append(abv[:, r * MSG:(r + 1) * MSG] / d)
    e = jnp.maximum(jnp.concatenate(parts, axis=1), 0.0)
    e = jnp.maximum(jnp.dot(e, wg[...], preferred_element_type=jnp.float32) + bg[...], 0.0)
    e = jnp.dot(e, wl[...], preferred_element_type=jnp.float32) + bl[...]
    out[...] = jnp.concatenate(
        [e, jnp.zeros((e.shape[0], LANE - EMBED), jnp.float32)], axis=1)


def _post(ab, wg, bg, wl, bl):
    n = ab.shape[0]
    full = lambda a: pl.BlockSpec(a.shape, lambda i: (0,) * a.ndim)
    args = [ab, wg, bg.reshape(1, -1), wl, bl.reshape(1, -1)]
    in_specs = [pl.BlockSpec((BM, LANE), lambda i: (i, 0))]
    in_specs += [full(a) for a in args[1:]]
    return pl.pallas_call(
        _post_body,
        grid=(n // BM,),
        in_specs=in_specs,
        out_specs=pl.BlockSpec((BM, LANE), lambda i: (i, 0)),
        out_shape=jax.ShapeDtypeStruct((n, LANE), jnp.float32),
    )(*args)


def _decode_body(uu, ii, qf, em, out):
    t = jnp.dot(uu[...].astype(jnp.bfloat16), qf[...],
                preferred_element_type=jnp.float32)
    iiv = ii[...][:, :EMBED]
    m = t * jnp.concatenate([iiv] * R, axis=1)
    out[...] = jnp.dot(m.astype(jnp.bfloat16), em[...],
                       preferred_element_type=jnp.float32)


def _decode(uu, ii, qf, em):
    full = lambda a: pl.BlockSpec(a.shape, lambda i: (0,) * a.ndim)
    return pl.pallas_call(
        _decode_body,
        grid=(pl.cdiv(P, BMD),),
        in_specs=[pl.BlockSpec((BMD, LANE), lambda i: (i, 0)),
                  pl.BlockSpec((BMD, LANE), lambda i: (i, 0)),
                  full(qf), full(em)],
        out_specs=pl.BlockSpec((BMD, R), lambda i: (i, 0)),
        out_shape=jax.ShapeDtypeStruct((P, R), jnp.float32),
    )(uu, ii, qf, em)


def _enc_sc_body(up0, up1, up2, up3, up4, ip0, ip1, ip2, ip3, ip4, enc5,
                 outu, outi,
                 asp0, asp1, asp2, asp3, asp4, dsp,
                 zbuf, idxd, idxs, rows, ones, gsem, ssem):
    c = lax.axis_index("c")
    s = lax.axis_index("s")
    asps = (asp0, asp1, asp2, asp3, asp4)

    def zrow(i, _):
        zbuf[i, :] = jnp.zeros((MSG,), jnp.float32)
        return 0

    lax.fori_loop(0, NODES_PT, zrow, 0)
    for sp in asps + (dsp,):
        @pl.when(s < NS - 1)
        def _():
            pltpu.sync_copy(zbuf, sp.at[pl.ds(s * NODES_PT, NODES_PT)])

        @pl.when(s == NS - 1)
        def _():
            pltpu.sync_copy(zbuf.at[pl.ds(0, NODES_LAST)],
                            sp.at[pl.ds((NS - 1) * NODES_PT, NODES_LAST)])
    plsc.subcore_barrier()

    def accumulate(dst_row, src_row, tabs):
        for r in range(R):
            onehot = jnp.where(lax.iota(jnp.int32, MSG) == r, 1.0, 0.0).astype(jnp.float32)

            def orow(i, _):
                ones[i, :] = onehot
                return 0

            lax.fori_loop(0, CH, orow, 0)
            pltpu.sync_copy(enc5.at[dst_row, r, s], idxd)
            pltpu.sync_copy(enc5.at[src_row, r, s], idxs)
            tab = tabs[r]
            asp = asps[r]
            for b in range(RING // 2):
                pltpu.async_copy(tab.at[idxs.at[b]], rows.at[b], gsem.at[b])

            def group(kk, _):
                for b in range(RING):
                    j = kk * RING + b
                    bn = (b + RING // 2) % RING

                    @pl.when(j >= RING // 2)
                    def _():
                        pltpu.make_async_copy(
                            rows.at[bn], asp.at[idxd.at[j - RING // 2]],
                            ssem.at[bn]).wait()
                        pltpu.make_async_copy(
                            ones, dsp.at[idxd.at[j - RING // 2]],
                            ssem.at[bn]).wait()

                    @pl.when(j + RING // 2 < NCH)
                    def _():
                        pltpu.async_copy(tab.at[idxs.at[j + RING // 2]],
                                         rows.at[bn], gsem.at[bn])

                    pltpu.make_async_copy(tab.at[idxs.at[j]], rows.at[b],
                                          gsem.at[b]).wait()
                    pltpu.async_copy(rows.at[b], asp.at[idxd.at[j]],
                                     ssem.at[b], add=True)
                    pltpu.async_copy(ones, dsp.at[idxd.at[j]],
                                     ssem.at[b], add=True)
                return 0

            lax.fori_loop(0, NCH // RING, group, 0)
            for b in range(RING // 2, RING):
                j = NCH - RING + b
                pltpu.make_async_copy(rows.at[b], asp.at[idxd.at[j]],
                                      ssem.at[b]).wait()
                pltpu.make_async_copy(ones, dsp.at[idxd.at[j]],
                                      ssem.at[b]).wait()

    @pl.when(c == 0)
    def _():
        accumulate(0, 1, (ip0, ip1, ip2, ip3, ip4))

    @pl.when(c == 1)
    def _():
        accumulate(1, 0, (up0, up1, up2, up3, up4))

    plsc.subcore_barrier()

    for cc, o in ((0, outu), (1, outi)):
        @pl.when((c == cc) & (s < NS - 1))
        def _():
            sl = pl.ds(s * NODES_PT, NODES_PT)
            for r, sp in enumerate(asps):
                pltpu.sync_copy(sp.at[sl], o.at[sl, pl.ds(r * MSG, MSG)])
            pltpu.sync_copy(dsp.at[sl], o.at[sl, pl.ds(DEG0, MSG)])

        @pl.when((c == cc) & (s == NS - 1))
        def _():
            sl = pl.ds((NS - 1) * NODES_PT, NODES_LAST)
            for r, sp in enumerate(asps):
                pltpu.sync_copy(sp.at[sl], o.at[sl, pl.ds(r * MSG, MSG)])
            pltpu.sync_copy(dsp.at[sl], o.at[sl, pl.ds(DEG0, MSG)])


def _enc_sc(ups, ips, enc5):
    mesh = plsc.VectorSubcoreMesh(core_axis_name="c", subcore_axis_name="s",
                                  num_cores=NC, num_subcores=NS)
    out_type = [jax.ShapeDtypeStruct((N_U, LANE), jnp.float32)] * 2
    scratch = ([pltpu.VMEM_SHARED((N_U, MSG), jnp.float32)] * R
               + [pltpu.VMEM_SHARED((N_U, MSG), jnp.float32)]
               + [pltpu.VMEM((NODES_PT, MSG), jnp.float32),
                  pltpu.VMEM((NCH, CH), jnp.int32),
                  pltpu.VMEM((NCH, CH), jnp.int32),
                  pltpu.VMEM((RING, CH, MSG), jnp.float32),
                  pltpu.VMEM((CH, MSG), jnp.float32),
                  pltpu.SemaphoreType.DMA((RING,)),
                  pltpu.SemaphoreType.DMA((RING,))])
    f = pl.kernel(_enc_sc_body, out_type=out_type, mesh=mesh,
                  scratch_types=scratch,
                  compiler_params=pltpu.CompilerParams(use_tc_tiling_on_sc=False))
    return f(*ups, *ips, enc5)


DCH = 80
DEC_ROWS = 1280
P_PAD = DEC_ROWS * DCH
DEC_RPT = DEC_ROWS // (NC * NS)
DRING = 4


def _dec_sc_body(u_emb, i_emb, dec3, uu, ii,
                 idxu, idxi, rub, rib, gsu, gsi, wsu, wsi):
    c = lax.axis_index("c")
    s = lax.axis_index("s")
    w = s * NC + c
    pltpu.sync_copy(dec3.at[0, pl.ds(w * DEC_RPT, DEC_RPT)], idxu)
    pltpu.sync_copy(dec3.at[1, pl.ds(w * DEC_RPT, DEC_RPT)], idxi)
    for b in range(DRING // 2):
        pltpu.async_copy(u_emb.at[idxu.at[b]], rub.at[b], gsu.at[b])
        pltpu.async_copy(i_emb.at[idxi.at[b]], rib.at[b], gsi.at[b])

    def group(kk, _):
        for b in range(DRING):
            j = kk * DRING + b
            bn = (b + DRING // 2) % DRING
            jj = w * DEC_RPT + j

            @pl.when(j >= DRING // 2)
            def _():
                jo = (w * DEC_RPT + j - DRING // 2) * DCH
                pltpu.make_async_copy(rub.at[bn], uu.at[pl.ds(jo, DCH)],
                                      wsu.at[bn]).wait()
                pltpu.make_async_copy(rib.at[bn], ii.at[pl.ds(jo, DCH)],
                                      wsi.at[bn]).wait()

            @pl.when(j + DRING // 2 < DEC_RPT)
            def _():
                pltpu.async_copy(u_emb.at[idxu.at[j + DRING // 2]],
                                 rub.at[bn], gsu.at[bn])
                pltpu.async_copy(i_emb.at[idxi.at[j + DRING // 2]],
                                 rib.at[bn], gsi.at[bn])

            pltpu.make_async_copy(u_emb.at[idxu.at[j]], rub.at[b],
                                  gsu.at[b]).wait()
            pltpu.make_async_copy(i_emb.at[idxi.at[j]], rib.at[b],
                                  gsi.at[b]).wait()
            pltpu.async_copy(rub.at[b], uu.at[pl.ds(jj * DCH, DCH)],
                             wsu.at[b])
            pltpu.async_copy(rib.at[b], ii.at[pl.ds(jj * DCH, DCH)],
                             wsi.at[b])
        return 0

    lax.fori_loop(0, DEC_RPT // DRING, group, 0)
    for b in range(DRING // 2, DRING):
        jo = (w * DEC_RPT + DEC_RPT - DRING + b) * DCH
        pltpu.make_async_copy(rub.at[b], uu.at[pl.ds(jo, DCH)],
                              wsu.at[b]).wait()
        pltpu.make_async_copy(rib.at[b], ii.at[pl.ds(jo, DCH)],
                              wsi.at[b]).wait()


def _dec_sc(u_emb, i_emb, dec3):
    mesh = plsc.VectorSubcoreMesh(core_axis_name="c", subcore_axis_name="s",
                                  num_cores=NC, num_subcores=NS)
    out_type = [jax.ShapeDtypeStruct((P_PAD, LANE), jnp.float32)] * 2
    scratch = [pltpu.VMEM((DEC_RPT, DCH), jnp.int32),
               pltpu.VMEM((DEC_RPT, DCH), jnp.int32),
               pltpu.VMEM((DRING, DCH, LANE), jnp.float32),
               pltpu.VMEM((DRING, DCH, LANE), jnp.float32),
               pltpu.SemaphoreType.DMA((DRING,)),
               pltpu.SemaphoreType.DMA((DRING,)),
               pltpu.SemaphoreType.DMA((DRING,)),
               pltpu.SemaphoreType.DMA((DRING,))]
    f = pl.kernel(_dec_sc_body, out_type=out_type, mesh=mesh,
                  scratch_types=scratch,
                  compiler_params=pltpu.CompilerParams(use_tc_tiling_on_sc=False))
    return f(u_emb, i_emb, dec3)


def kernel(u_feat, i_feat, uW0, ub0, uW1, ub1, uW2, ub2, uW3, ub3,
           iW0, ib0, iW1, ib1, iW2, ib2, iW3, ib3,
           Wr, Wgu, bgu, Wgi, bgi, Wu, bu, Wi, bi, Q,
           enc_edges, dec_edges):
    enc5 = enc_edges.astype(jnp.int32).reshape(2, R, NS, NCH, CH)
    dec_pad = jnp.concatenate(
        [dec_edges.astype(jnp.int32),
         jnp.zeros((2, P_PAD - P), jnp.int32)], axis=1)
    dec3 = dec_pad.reshape(2, DEC_ROWS, DCH)
    wrf = jnp.transpose(Wr, (1, 0, 2)).reshape(EMBED, R * MSG)
    qf = jnp.concatenate(
        [jnp.transpose(Q, (1, 0, 2)).reshape(EMBED, R * EMBED),
         jnp.zeros((LANE - EMBED, R * EMBED), jnp.float32)],
        axis=0).astype(jnp.bfloat16)
    em = (jnp.arange(R * EMBED)[:, None] // EMBED
          == jnp.arange(R)[None, :]).astype(jnp.bfloat16)

    ups = _mlp_proj(u_feat, (uW0, uW1, uW2, uW3), (ub0, ub1, ub2, ub3), wrf)
    ips = _mlp_proj(i_feat, (iW0, iW1, iW2, iW3), (ib0, ib1, ib2, ib3), wrf)

    outu, outi = _enc_sc(ups, ips, enc5)
    u_emb = _post(outu, Wgu, bgu, Wu, bu)
    i_emb = _post(outi, Wgi, bgi, Wi, bi)

    uu, ii = _dec_sc(u_emb, i_emb, dec3)
    return _decode(uu, ii, qf, em)


# speedup vs baseline: 12.7550x; 1.3125x over previous
import functools

import jax
import jax.numpy as jnp
from jax import lax
from jax.experimental import pallas as pl
from jax.experimental.pallas import tpu as pltpu
from jax.experimental.pallas import tpu_sc as plsc

N_U = 10000
N_I = 10000
EMBED = 80
R = 5
MSG = EMBED // R
E_TOTAL = 640000
E_PER = E_TOTAL // R
P = 100000

NC = 2
NS = 16
EPT = E_PER // NS
CH = 100
NCH = EPT // CH
RING = 8
NODES_PT = 640
NODES_LAST = N_U - NODES_PT * (NS - 1)

BM = 1000
BMD = 2048
LANE = 128
DEG0 = EMBED


def _mlp_proj_body(x, w0, b0, w1, b1, w2, b2, w3, b3, wrf,
                   o0, o1, o2, o3, o4):
    h = jnp.maximum(jnp.dot(x[...], w0[...], preferred_element_type=jnp.float32) + b0[...], 0.0)
    h = jnp.maximum(jnp.dot(h, w1[...], preferred_element_type=jnp.float32) + b1[...], 0.0)
    h = jnp.maximum(jnp.dot(h, w2[...], preferred_element_type=jnp.float32) + b2[...], 0.0)
    h = jnp.maximum(jnp.dot(h, w3[...], preferred_element_type=jnp.float32) + b3[...], 0.0)
    p = jnp.dot(h, wrf[...], preferred_element_type=jnp.float32)
    for r, o in enumerate((o0, o1, o2, o3, o4)):
        o[...] = p[:, r * MSG:(r + 1) * MSG]


def _mlp_proj(x, ws, bs, wrf):
    n = x.shape[0]
    full = lambda a: pl.BlockSpec(a.shape, lambda i: (0,) * a.ndim)
    in_specs = [pl.BlockSpec((BM, x.shape[1]), lambda i: (i, 0))]
    args = [x]
    for w, b in zip(ws, bs):
        args += [w, b.reshape(1, -1)]
    args.append(wrf)
    in_specs += [full(a) for a in args[1:]]
    return pl.pallas_call(
        _mlp_proj_body,
        grid=(n // BM,),
        in_specs=in_specs,
        out_specs=[pl.BlockSpec((BM, MSG), lambda i: (i, 0))] * R,
        out_shape=[jax.ShapeDtypeStruct((n, MSG), jnp.float32)] * R,
    )(*args)


def _post_body(ab, wg, bg, wl, bl, out):
    abv = ab[...]
    parts = []
    for r in range(R):
        d = jnp.maximum(abv[:, DEG0 + r:DEG0 + r + 1], 1.0)
        parts.append(abv[:, r * MSG:(r + 1) * MSG] / d)
    e = jnp.maximum(jnp.concatenate(parts, axis=1), 0.0)
    e = jnp.maximum(jnp.dot(e, wg[...], preferred_element_type=jnp.float32) + bg[...], 0.0)
    out[...] = jnp.dot(e, wl[...], preferred_element_type=jnp.float32) + bl[...]


def _post(ab, wg, bg, wl, bl):
    n = ab.shape[0]
    full = lambda a: pl.BlockSpec(a.shape, lambda i: (0,) * a.ndim)
    args = [ab, wg, bg.reshape(1, -1), wl, bl.reshape(1, -1)]
    in_specs = [pl.BlockSpec((BM, LANE), lambda i: (i, 0))]
    in_specs += [full(a) for a in args[1:]]
    return pl.pallas_call(
        _post_body,
        grid=(n // BM,),
        in_specs=in_specs,
        out_specs=pl.BlockSpec((BM, EMBED), lambda i: (i, 0)),
        out_shape=jax.ShapeDtypeStruct((n, EMBED), jnp.float32),
    )(*args)


def _decode_body(uu, ii, qf, em, out):
    t = jnp.dot(uu[...][:, :EMBED].astype(jnp.bfloat16), qf[...],
                preferred_element_type=jnp.float32)
    iiv = ii[...][:, :EMBED]
    m = t * jnp.concatenate([iiv] * R, axis=1)
    out[...] = jnp.dot(m.astype(jnp.bfloat16), em[...],
                       preferred_element_type=jnp.float32)


def _decode(uu, ii, qf, em):
    full = lambda a: pl.BlockSpec(a.shape, lambda i: (0,) * a.ndim)
    return pl.pallas_call(
        _decode_body,
        grid=(pl.cdiv(P, BMD),),
        in_specs=[pl.BlockSpec((BMD, LANE), lambda i: (i, 0)),
                  pl.BlockSpec((BMD, LANE), lambda i: (i, 0)),
                  full(qf), full(em)],
        out_specs=pl.BlockSpec((BMD, R), lambda i: (i, 0)),
        out_shape=jax.ShapeDtypeStruct((P, R), jnp.float32),
    )(uu, ii, qf, em)


def _enc_sc_body(up0, up1, up2, up3, up4, ip0, ip1, ip2, ip3, ip4, enc5,
                 outu, outi,
                 asp0, asp1, asp2, asp3, asp4, dsp,
                 zbuf, idxd, idxs, rows, ones, gsem, ssem):
    c = lax.axis_index("c")
    s = lax.axis_index("s")
    asps = (asp0, asp1, asp2, asp3, asp4)

    def zrow(i, _):
        zbuf[i, :] = jnp.zeros((MSG,), jnp.float32)
        return 0

    lax.fori_loop(0, NODES_PT, zrow, 0)
    for sp in asps + (dsp,):
        @pl.when(s < NS - 1)
        def _():
            pltpu.sync_copy(zbuf, sp.at[pl.ds(s * NODES_PT, NODES_PT)])

        @pl.when(s == NS - 1)
        def _():
            pltpu.sync_copy(zbuf.at[pl.ds(0, NODES_LAST)],
                            sp.at[pl.ds((NS - 1) * NODES_PT, NODES_LAST)])
    plsc.subcore_barrier()

    def accumulate(dst_row, src_row, tabs):
        for r in range(R):
            onehot = jnp.where(lax.iota(jnp.int32, MSG) == r, 1.0, 0.0).astype(jnp.float32)

            def orow(i, _):
                ones[i, :] = onehot
                return 0

            lax.fori_loop(0, CH, orow, 0)
            pltpu.sync_copy(enc5.at[dst_row, r, s], idxd)
            pltpu.sync_copy(enc5.at[src_row, r, s], idxs)
            tab = tabs[r]
            asp = asps[r]
            for b in range(RING // 2):
                pltpu.async_copy(tab.at[idxs.at[b]], rows.at[b], gsem.at[b])

            def group(kk, _):
                for b in range(RING):
                    j = kk * RING + b
                    bn = (b + RING // 2) % RING

                    @pl.when(j >= RING // 2)
                    def _():
                        pltpu.make_async_copy(
                            rows.at[bn], asp.at[idxd.at[j - RING // 2]],
                            ssem.at[bn]).wait()
                        pltpu.make_async_copy(
                            ones, dsp.at[idxd.at[j - RING // 2]],
                            ssem.at[bn]).wait()

                    @pl.when(j + RING // 2 < NCH)
                    def _():
                        pltpu.async_copy(tab.at[idxs.at[j + RING // 2]],
                                         rows.at[bn], gsem.at[bn])

                    pltpu.make_async_copy(tab.at[idxs.at[j]], rows.at[b],
                                          gsem.at[b]).wait()
                    pltpu.async_copy(rows.at[b], asp.at[idxd.at[j]],
                                     ssem.at[b], add=True)
                    pltpu.async_copy(ones, dsp.at[idxd.at[j]],
                                     ssem.at[b], add=True)
                return 0

            lax.fori_loop(0, NCH // RING, group, 0)
            for b in range(RING // 2, RING):
                j = NCH - RING + b
                pltpu.make_async_copy(rows.at[b], asp.at[idxd.at[j]],
                                      ssem.at[b]).wait()
                pltpu.make_async_copy(ones, dsp.at[idxd.at[j]],
                                      ssem.at[b]).wait()

    @pl.when(c == 0)
    def _():
        accumulate(0, 1, (ip0, ip1, ip2, ip3, ip4))

    @pl.when(c == 1)
    def _():
        accumulate(1, 0, (up0, up1, up2, up3, up4))

    plsc.subcore_barrier()

    for cc, o in ((0, outu), (1, outi)):
        @pl.when((c == cc) & (s < NS - 1))
        def _():
            sl = pl.ds(s * NODES_PT, NODES_PT)
            for r, sp in enumerate(asps):
                pltpu.sync_copy(sp.at[sl], o.at[sl, pl.ds(r * MSG, MSG)])
            pltpu.sync_copy(dsp.at[sl], o.at[sl, pl.ds(DEG0, MSG)])

        @pl.when((c == cc) & (s == NS - 1))
        def _():
            sl = pl.ds((NS - 1) * NODES_PT, NODES_LAST)
            for r, sp in enumerate(asps):
                pltpu.sync_copy(sp.at[sl], o.at[sl, pl.ds(r * MSG, MSG)])
            pltpu.sync_copy(dsp.at[sl], o.at[sl, pl.ds(DEG0, MSG)])


def _enc_sc(ups, ips, enc5):
    mesh = plsc.VectorSubcoreMesh(core_axis_name="c", subcore_axis_name="s",
                                  num_cores=NC, num_subcores=NS)
    out_type = [jax.ShapeDtypeStruct((N_U, LANE), jnp.float32)] * 2
    scratch = ([pltpu.VMEM_SHARED((N_U, MSG), jnp.float32)] * R
               + [pltpu.VMEM_SHARED((N_U, MSG), jnp.float32)]
               + [pltpu.VMEM((NODES_PT, MSG), jnp.float32),
                  pltpu.VMEM((NCH, CH), jnp.int32),
                  pltpu.VMEM((NCH, CH), jnp.int32),
                  pltpu.VMEM((RING, CH, MSG), jnp.float32),
                  pltpu.VMEM((CH, MSG), jnp.float32),
                  pltpu.SemaphoreType.DMA((RING,)),
                  pltpu.SemaphoreType.DMA((RING,))])
    f = pl.kernel(_enc_sc_body, out_type=out_type, mesh=mesh,
                  scratch_types=scratch,
                  compiler_params=pltpu.CompilerParams(use_tc_tiling_on_sc=False))
    return f(*ups, *ips, enc5)


DCH = 80
DEC_ROWS = P // DCH
DEC_RPT = 80
DRING = 4


def _dec_sc_body(u_emb, i_emb, dec3, uu, ii,
                 sp, idx, rb, gs, ws):
    c = lax.axis_index("c")
    s = lax.axis_index("s")

    def side(emb, dec_row, out):
        @pl.when(s < NS - 1)
        def _():
            sl = pl.ds(s * NODES_PT, NODES_PT)
            pltpu.sync_copy(emb.at[sl], sp.at[sl])

        @pl.when(s == NS - 1)
        def _():
            sl = pl.ds((NS - 1) * NODES_PT, NODES_LAST)
            pltpu.sync_copy(emb.at[sl], sp.at[sl])

        plsc.subcore_barrier()

        @pl.when(s < NS - 1)
        def _():
            pltpu.sync_copy(dec3.at[dec_row, pl.ds(s * DEC_RPT, DEC_RPT)], idx)

        @pl.when(s == NS - 1)
        def _():
            nlast = DEC_ROWS - (NS - 1) * DEC_RPT
            pltpu.sync_copy(dec3.at[dec_row, pl.ds((NS - 1) * DEC_RPT, nlast)],
                            idx.at[pl.ds(0, nlast)])

        for b in range(DRING // 2):
            pltpu.async_copy(sp.at[idx.at[b]], rb.at[b], gs.at[b])

        def group(kk, _):
            for b in range(DRING):
                j = kk * DRING + b
                bn = (b + DRING // 2) % DRING
                jj = s * DEC_RPT + j

                @pl.when((j >= DRING // 2) & (jj - DRING // 2 < DEC_ROWS))
                def _():
                    jo = (jj - DRING // 2) * DCH
                    pltpu.make_async_copy(
                        rb.at[bn], out.at[pl.ds(jo, DCH), pl.ds(0, EMBED)],
                        ws.at[bn]).wait()

                @pl.when((j + DRING // 2 < DEC_RPT)
                         & (jj + DRING // 2 < DEC_ROWS))
                def _():
                    pltpu.async_copy(sp.at[idx.at[j + DRING // 2]],
                                     rb.at[bn], gs.at[bn])

                @pl.when(jj < DEC_ROWS)
                def _():
                    pltpu.make_async_copy(sp.at[idx.at[j]], rb.at[b],
                                          gs.at[b]).wait()
                    pltpu.async_copy(
                        rb.at[b], out.at[pl.ds(jj * DCH, DCH), pl.ds(0, EMBED)],
                        ws.at[b])
            return 0

        lax.fori_loop(0, DEC_RPT // DRING, group, 0)

        @pl.when(s < NS - 1)
        def _():
            for b in range(DRING // 2, DRING):
                jo = (s * DEC_RPT + DEC_RPT - DRING + b) * DCH
                pltpu.make_async_copy(
                    rb.at[b], out.at[pl.ds(jo, DCH), pl.ds(0, EMBED)],
                    ws.at[b]).wait()

    @pl.when(c == 0)
    def _():
        side(u_emb, 0, uu)

    @pl.when(c == 1)
    def _():
        side(i_emb, 1, ii)


def _dec_sc(u_emb, i_emb, dec3):
    mesh = plsc.VectorSubcoreMesh(core_axis_name="c", subcore_axis_name="s",
                                  num_cores=NC, num_subcores=NS)
    out_type = [jax.ShapeDtypeStruct((P, LANE), jnp.float32)] * 2
    scratch = [pltpu.VMEM_SHARED((N_U, EMBED), jnp.float32),
               pltpu.VMEM((DEC_RPT, DCH), jnp.int32),
               pltpu.VMEM((DRING, DCH, EMBED), jnp.float32),
               pltpu.SemaphoreType.DMA((DRING,)),
               pltpu.SemaphoreType.DMA((DRING,))]
    f = pl.kernel(_dec_sc_body, out_type=out_type, mesh=mesh,
                  scratch_types=scratch,
                  compiler_params=pltpu.CompilerParams(use_tc_tiling_on_sc=False))
    return f(u_emb, i_emb, dec3)


def kernel(u_feat, i_feat, uW0, ub0, uW1, ub1, uW2, ub2, uW3, ub3,
           iW0, ib0, iW1, ib1, iW2, ib2, iW3, ib3,
           Wr, Wgu, bgu, Wgi, bgi, Wu, bu, Wi, bi, Q,
           enc_edges, dec_edges):
    enc5 = enc_edges.astype(jnp.int32).reshape(2, R, NS, NCH, CH)
    dec3 = dec_edges.astype(jnp.int32).reshape(2, DEC_ROWS, DCH)
    wrf = jnp.transpose(Wr, (1, 0, 2)).reshape(EMBED, R * MSG)
    qf = jnp.transpose(Q, (1, 0, 2)).reshape(EMBED, R * EMBED).astype(jnp.bfloat16)
    em = (jnp.arange(R * EMBED)[:, None] // EMBED
          == jnp.arange(R)[None, :]).astype(jnp.bfloat16)

    ups = _mlp_proj(u_feat, (uW0, uW1, uW2, uW3), (ub0, ub1, ub2, ub3), wrf)
    ips = _mlp_proj(i_feat, (iW0, iW1, iW2, iW3), (ib0, ib1, ib2, ib3), wrf)

    outu, outi = _enc_sc(ups, ips, enc5)
    u_emb = _post(outu, Wgu, bgu, Wu, bu)
    i_emb = _post(outi, Wgi, bgi, Wi, bi)

    uu, ii = _dec_sc(u_emb, i_emb, dec3)
    return _decode(uu, ii, qf, em)
